# baseline (speedup 1.0000x reference)
import functools

import numpy as np
import jax
import jax.numpy as jnp
from jax import lax
from jax.experimental import pallas as pl
from jax.experimental.pallas import tpu as pltpu
from jax.experimental.pallas import tpu_sc as plsc

B = 1024
D = 64
N = 100000
CB = 2048
NBLK = 49
NP = NBLK * CB
W = 16
WPB = CB // W
NWIN = NP // W
GPB = WPB // 16
NGRP = NBLK * GPB
NGP = 400
K_NB = 50
N_CLS = 10
NEG = -3.0
RB = 256
IDM = 0x1FFF


def _tree16(vals, op):
    while len(vals) > 1:
        vals = [op(vals[i], vals[i + 1]) if i + 1 < len(vals) else vals[i]
                for i in range(0, len(vals), 2)]
    return vals[0]


def _i32c(v):
    return jnp.int32(np.int32(np.uint32(v)))


def _tc_body(x_ref, k_ref, s_ref, mp_ref, m2_ref):
    blk = pl.program_id(1)
    x = x_ref[...]
    ss = jnp.sum(x * x, axis=1, keepdims=True)
    xn = (x / jnp.maximum(jnp.sqrt(ss), 1e-12)).astype(jnp.bfloat16)
    kb = k_ref[...].astype(jnp.bfloat16)
    s = jax.lax.dot_general(
        xn, kb, (((1,), (1,)), ((), ())),
        preferred_element_type=jnp.float32,
    )
    col = jax.lax.broadcasted_iota(jnp.int32, (RB, CB), 1) + blk * CB
    s = jnp.where(col >= N, NEG, s)
    s_ref[...] = s
    st = jax.lax.dot_general(
        kb, xn, (((1,), (1,)), ((), ())),
        preferred_element_type=jnp.float32,
    )
    row = jax.lax.broadcasted_iota(jnp.int32, (CB, RB), 0) + blk * CB
    st = jnp.where(row >= N, NEG, st)
    mt = jnp.max(st.reshape(WPB, W, RB), axis=1)
    u = jax.lax.bitcast_convert_type(mt, jnp.int32)
    key = jnp.where(u >= 0, u, u ^ _i32c(0x7FFFFFFF))
    gwin = (jax.lax.broadcasted_iota(jnp.int32, (WPB, RB), 0) + blk * WPB)
    mp = (key & _i32c(0xFFFFE000)) | gwin
    mp_ref[...] = jnp.swapaxes(mp, 0, 1)
    m2 = jnp.max(mp.reshape(GPB, 16, RB), axis=1)
    m2_ref[...] = jnp.swapaxes(m2, 0, 1)[None]


def _tc_scores(xf, keys_pad):
    return pl.pallas_call(
        _tc_body,
        grid=(B // RB, NBLK),
        in_specs=[
            pl.BlockSpec((RB, D), lambda r, c: (r, 0)),
            pl.BlockSpec((CB, D), lambda r, c: (c, 0)),
        ],
        out_specs=[
            pl.BlockSpec((RB, CB), lambda r, c: (r, c)),
            pl.BlockSpec((RB, WPB), lambda r, c: (r, c)),
            pl.BlockSpec((1, RB, GPB), lambda r, c: (c, r, 0)),
        ],
        out_shape=[
            jax.ShapeDtypeStruct((B, NP), jnp.float32),
            jax.ShapeDtypeStruct((B, NWIN), jnp.int32),
            jax.ShapeDtypeStruct((NBLK, B, GPB), jnp.int32),
        ],
    )(xf, keys_pad)


NSEL = 64
NWORK = 32
ROWS_PW = B // NWORK


def _sc_make():
    mesh = plsc.VectorSubcoreMesh(core_axis_name="c", subcore_axis_name="s")

    @functools.partial(
        pl.kernel,
        mesh=mesh,
        out_type=jax.ShapeDtypeStruct((B, 16), jnp.float32),
        scratch_types=[
            pltpu.VMEM((NGP // 16, 16), jnp.int32),
            pltpu.VMEM((NSEL, 16), jnp.int32),
            pltpu.VMEM((NSEL // 16, 16), jnp.int32),
            pltpu.VMEM((NSEL, 16), jnp.int32),
            pltpu.VMEM((NSEL, 16), jnp.float32),
            pltpu.VMEM((NSEL, 16), jnp.int32),
            pltpu.VMEM((NSEL, 16), jnp.float32),
            pltpu.VMEM((16,), jnp.float32),
            pltpu.SemaphoreType.DMA,
            pltpu.SemaphoreType.DMA,
        ],
    )
    def sck(m2_hbm, mp_hbm, s_hbm, v2_hbm, out_hbm, m2buf, exp, wids,
            gsplat, cand_s, cand_l, kwork, outrow, sem1, sem2):
        wid = lax.axis_index("s") * 2 + lax.axis_index("c")
        li16 = lax.iota(jnp.int32, 16)
        zi = jnp.zeros((16,), jnp.int32)
        zf = jnp.zeros((16,), jnp.float32)
        imin = _i32c(0x80000000)
        iminsp = jnp.full((16,), imin, jnp.int32)
        m7f = _i32c(0x7FFFFFFF)
        big = jnp.full((16,), m7f, jnp.int32)
        negsp = jnp.full((16,), jnp.float32(-4.0))

        def _extract16(buf, nv):
            def it_body(it, ids):
                parts = []
                for c0 in range(0, nv, 16):
                    rows = [buf[i] for i in range(c0, min(c0 + 16, nv))]
                    parts.append(_tree16(rows, jnp.maximum))
                mx = _tree16(parts, jnp.maximum)
                m_s = _tree16([mx[j] for j in range(16)], jnp.maximum)
                msp = jnp.full((16,), m_s, jnp.int32)
                for i in range(nv):
                    row = buf[i]
                    buf[i] = jnp.where(row == msp, iminsp, row)
                itsp = jnp.full((16,), it, jnp.int32)
                return jnp.where(li16 == itsp,
                                 jnp.full((16,), m_s & IDM, jnp.int32), ids)
            def quad(itq, ids):
                for k in range(4):
                    ids = it_body(itq * 4 + k, ids)
                return ids
            return lax.fori_loop(0, 4, quad, zi)

        def per_row(rr, _carry):
            r = wid * ROWS_PW + rr
            pltpu.sync_copy(m2_hbm.at[r], m2buf)

            for c in range(NSEL // 16):
                ids = _extract16(m2buf, NGP // 16)
                cps = []
                for l in range(16):
                    grp = lax.shift_right_logical(ids[l], jnp.int32(4))
                    cps.append(pltpu.async_copy(
                        mp_hbm.at[r, grp], exp.at[c * 16 + l], sem1))
                for cp in cps:
                    cp.wait()

            for c in range(NSEL // 16):
                ids = _extract16(exp, NSEL)
                wids[c] = ids
                cps = []
                for l in range(16):
                    g = ids[l]
                    cps.append(pltpu.async_copy(
                        s_hbm.at[r, g], cand_s.at[c * 16 + l], sem1))
                    cps.append(pltpu.async_copy(
                        v2_hbm.at[g], cand_l.at[c * 16 + l], sem2))
                for cp in cps:
                    cp.wait()

            for c in range(NSEL // 16):
                wv = wids[c]
                for l in range(16):
                    i = c * 16 + l
                    gsplat[i] = (jnp.full((16,), wv[l] * W, jnp.int32) + li16)
                    kwork[i] = cand_s[i]

            def sel_body(it, accs):
                parts = []
                for c0 in range(0, NSEL, 16):
                    rows = [kwork[i] for i in range(c0, c0 + 16)]
                    parts.append(_tree16(rows, jnp.maximum))
                mx = _tree16(parts, jnp.maximum)
                m_s = _tree16([mx[j] for j in range(16)], jnp.maximum)
                msp = jnp.full((16,), m_s, jnp.float32)

                parts = []
                for c0 in range(0, NSEL, 16):
                    pms = [jnp.where(kwork[i] == msp, gsplat[i], big)
                           for i in range(c0, c0 + 16)]
                    parts.append(_tree16(pms, jnp.minimum))
                pmv = _tree16(parts, jnp.minimum)
                p_s = _tree16([pmv[j] for j in range(16)], jnp.minimum)
                psp = jnp.full((16,), p_s, jnp.int32)

                labparts = []
                for c0 in range(0, NSEL, 16):
                    labs = []
                    for i in range(c0, c0 + 16):
                        row = kwork[i]
                        kill = jnp.logical_and(row == msp,
                                               gsplat[i] == psp)
                        labs.append(jnp.where(kill, cand_l[i], zi - 1))
                        kwork[i] = jnp.where(kill, negsp, row)
                    labparts.append(_tree16(labs, jnp.maximum))
                labv = _tree16(labparts, jnp.maximum)
                l_s = _tree16([labv[j] for j in range(16)], jnp.maximum)

                sc_s = m_s
                return tuple(
                    accs[cc] + sc_s * (l_s == cc).astype(jnp.float32)
                    for cc in range(N_CLS))

            def sel2(it2, accs):
                accs = sel_body(it2 * 2, accs)
                return sel_body(it2 * 2 + 1, accs)
            accs = lax.fori_loop(0, K_NB // 2, sel2,
                                 tuple(jnp.float32(0.0)
                                       for _ in range(N_CLS)))
            acc = zf
            for cc in range(N_CLS):
                acc = acc + jnp.where(
                    li16 == cc, jnp.full((16,), accs[cc], jnp.float32),
                    jnp.float32(0.0))
            outrow[...] = acc
            pltpu.sync_copy(outrow, out_hbm.at[r])
            return 0

        lax.fori_loop(0, ROWS_PW, per_row, 0)

    return sck


_sc_select = _sc_make()


def kernel(x, keys, values):
    keys_pad = jnp.concatenate(
        [keys, jnp.zeros((NP - N, D), keys.dtype)], axis=0)
    s_store, mp, m2p3 = _tc_scores(x, keys_pad)
    m2p = jnp.swapaxes(m2p3, 0, 1).reshape(B, NGRP)

    m2_pad = jnp.concatenate(
        [m2p, jnp.full((B, NGP - NGRP), np.int32(-2147483648), jnp.int32)],
        axis=1)
    v_store = jnp.concatenate(
        [values, jnp.zeros((NP - N,), values.dtype)], axis=0)

    logits16 = _sc_select(
        m2_pad.reshape(B, NGP // 16, 16),
        mp.reshape(B, NGRP, 16),
        s_store.reshape(B, NWIN, 16),
        v_store.reshape(NWIN, 16),
    )
    return logits16[:, :N_CLS]


# speedup vs baseline: 1.0398x; 1.0398x over previous
import functools

import numpy as np
import jax
import jax.numpy as jnp
from jax import lax
from jax.experimental import pallas as pl
from jax.experimental.pallas import tpu as pltpu
from jax.experimental.pallas import tpu_sc as plsc

B = 1024
D = 64
N = 100000
CB = 2048
NBLK = 49
NP = NBLK * CB
W = 16
WPB = CB // W
NWIN = NP // W
GPB = WPB // 16
NGRP = NBLK * GPB
NGP = 400
K_NB = 50
N_CLS = 10
NEG = -3.0
RB = 256
IDM = 0x1FFF


def _tree16(vals, op):
    while len(vals) > 1:
        vals = [op(vals[i], vals[i + 1]) if i + 1 < len(vals) else vals[i]
                for i in range(0, len(vals), 2)]
    return vals[0]


def _i32c(v):
    return jnp.int32(np.int32(np.uint32(v)))


def _tc_body(x_ref, k_ref, s_ref, mp_ref, m2_ref):
    blk = pl.program_id(1)
    x = x_ref[...]
    ss = jnp.sum(x * x, axis=1, keepdims=True)
    xn = (x / jnp.maximum(jnp.sqrt(ss), 1e-12)).astype(jnp.bfloat16)
    kb = k_ref[...].astype(jnp.bfloat16)
    s = jax.lax.dot_general(
        xn, kb, (((1,), (1,)), ((), ())),
        preferred_element_type=jnp.float32,
    )
    col = jax.lax.broadcasted_iota(jnp.int32, (RB, CB), 1) + blk * CB
    s = jnp.where(col >= N, NEG, s)
    s_ref[...] = s
    st = jax.lax.dot_general(
        kb, xn, (((1,), (1,)), ((), ())),
        preferred_element_type=jnp.float32,
    )
    row = jax.lax.broadcasted_iota(jnp.int32, (CB, RB), 0) + blk * CB
    st = jnp.where(row >= N, NEG, st)
    mt = jnp.max(st.reshape(WPB, W, RB), axis=1)
    u = jax.lax.bitcast_convert_type(mt, jnp.int32)
    key = jnp.where(u >= 0, u, u ^ _i32c(0x7FFFFFFF))
    gwin = (jax.lax.broadcasted_iota(jnp.int32, (WPB, RB), 0) + blk * WPB)
    mp = (key & _i32c(0xFFFFE000)) | gwin
    mp_ref[...] = jnp.swapaxes(mp, 0, 1)
    m2 = jnp.max(mp.reshape(GPB, 16, RB), axis=1)
    m2_ref[...] = jnp.swapaxes(m2, 0, 1)[None]


def _tc_scores(xf, keys_pad):
    return pl.pallas_call(
        _tc_body,
        grid=(B // RB, NBLK),
        in_specs=[
            pl.BlockSpec((RB, D), lambda r, c: (r, 0)),
            pl.BlockSpec((CB, D), lambda r, c: (c, 0)),
        ],
        out_specs=[
            pl.BlockSpec((RB, CB), lambda r, c: (r, c)),
            pl.BlockSpec((RB, WPB), lambda r, c: (r, c)),
            pl.BlockSpec((1, RB, GPB), lambda r, c: (c, r, 0)),
        ],
        out_shape=[
            jax.ShapeDtypeStruct((B, NP), jnp.float32),
            jax.ShapeDtypeStruct((B, NWIN), jnp.int32),
            jax.ShapeDtypeStruct((NBLK, B, GPB), jnp.int32),
        ],
    )(xf, keys_pad)


NSEL = 64
NWORK = 32
ROWS_PW = B // NWORK


def _sc_make():
    mesh = plsc.VectorSubcoreMesh(core_axis_name="c", subcore_axis_name="s")

    @functools.partial(
        pl.kernel,
        mesh=mesh,
        out_type=jax.ShapeDtypeStruct((B, 16), jnp.float32),
        scratch_types=[
            pltpu.VMEM((NGP // 16, 16), jnp.int32),
            pltpu.VMEM((NSEL, 16), jnp.int32),
            pltpu.VMEM((NSEL // 16, 16), jnp.int32),
            pltpu.VMEM((NSEL, 16), jnp.int32),
            pltpu.VMEM((NSEL, 16), jnp.float32),
            pltpu.VMEM((NSEL, 16), jnp.int32),
            pltpu.VMEM((NSEL, 16), jnp.float32),
            pltpu.VMEM((16,), jnp.float32),
            pltpu.SemaphoreType.DMA,
            pltpu.SemaphoreType.DMA,
        ],
    )
    def sck(m2_hbm, mp_hbm, s_hbm, v2_hbm, out_hbm, m2buf, exp, wids,
            gsplat, cand_s, cand_l, kwork, outrow, sem1, sem2):
        wid = lax.axis_index("s") * 2 + lax.axis_index("c")
        li16 = lax.iota(jnp.int32, 16)
        zi = jnp.zeros((16,), jnp.int32)
        zf = jnp.zeros((16,), jnp.float32)
        imin = _i32c(0x80000000)
        iminsp = jnp.full((16,), imin, jnp.int32)
        m7f = _i32c(0x7FFFFFFF)
        big = jnp.full((16,), m7f, jnp.int32)
        negsp = jnp.full((16,), jnp.float32(-4.0))

        def _extract16(buf, nv):
            def it_body(it, ids):
                parts = []
                for c0 in range(0, nv, 16):
                    rows = [buf[i] for i in range(c0, min(c0 + 16, nv))]
                    parts.append(_tree16(rows, jnp.maximum))
                mx = _tree16(parts, jnp.maximum)
                m_s = _tree16([mx[j] for j in range(16)], jnp.maximum)
                msp = jnp.full((16,), m_s, jnp.int32)
                for i in range(nv):
                    row = buf[i]
                    buf[i] = jnp.where(row == msp, iminsp, row)
                itsp = jnp.full((16,), it, jnp.int32)
                return jnp.where(li16 == itsp,
                                 jnp.full((16,), m_s & IDM, jnp.int32), ids)
            return lax.fori_loop(0, 16, it_body, zi)

        def per_row(rr, _carry):
            r = wid * ROWS_PW + rr
            pltpu.sync_copy(m2_hbm.at[r], m2buf)

            for c in range(NSEL // 16):
                ids = _extract16(m2buf, NGP // 16)
                cps = []
                for l in range(16):
                    grp = lax.shift_right_logical(ids[l], jnp.int32(4))
                    cps.append(pltpu.async_copy(
                        mp_hbm.at[r, grp], exp.at[c * 16 + l], sem1))
                for cp in cps:
                    cp.wait()

            for c in range(NSEL // 16):
                ids = _extract16(exp, NSEL)
                wids[c] = ids
                cps = []
                for l in range(16):
                    g = ids[l]
                    cps.append(pltpu.async_copy(
                        s_hbm.at[r, g], cand_s.at[c * 16 + l], sem1))
                    cps.append(pltpu.async_copy(
                        v2_hbm.at[g], cand_l.at[c * 16 + l], sem2))
                for cp in cps:
                    cp.wait()

            for c in range(NSEL // 16):
                wv = wids[c]
                for l in range(16):
                    i = c * 16 + l
                    gsplat[i] = (jnp.full((16,), wv[l] * W, jnp.int32) + li16)
                    kwork[i] = cand_s[i]

            def sel_body(it, accs):
                parts = []
                for c0 in range(0, NSEL, 16):
                    rows = [kwork[i] for i in range(c0, c0 + 16)]
                    parts.append(_tree16(rows, jnp.maximum))
                mx = _tree16(parts, jnp.maximum)
                m_s = _tree16([mx[j] for j in range(16)], jnp.maximum)
                msp = jnp.full((16,), m_s, jnp.float32)

                parts = []
                for c0 in range(0, NSEL, 16):
                    pms = [jnp.where(kwork[i] == msp, gsplat[i], big)
                           for i in range(c0, c0 + 16)]
                    parts.append(_tree16(pms, jnp.minimum))
                pmv = _tree16(parts, jnp.minimum)
                p_s = _tree16([pmv[j] for j in range(16)], jnp.minimum)
                psp = jnp.full((16,), p_s, jnp.int32)

                labparts = []
                for c0 in range(0, NSEL, 16):
                    labs = []
                    for i in range(c0, c0 + 16):
                        row = kwork[i]
                        kill = jnp.logical_and(row == msp,
                                               gsplat[i] == psp)
                        labs.append(jnp.where(kill, cand_l[i], zi - 1))
                        kwork[i] = jnp.where(kill, negsp, row)
                    labparts.append(_tree16(labs, jnp.maximum))
                labv = _tree16(labparts, jnp.maximum)
                l_s = _tree16([labv[j] for j in range(16)], jnp.maximum)

                sc_s = m_s
                return tuple(
                    accs[cc] + sc_s * (l_s == cc).astype(jnp.float32)
                    for cc in range(N_CLS))

            accs = lax.fori_loop(0, K_NB, sel_body,
                                 tuple(jnp.float32(0.0)
                                       for _ in range(N_CLS)))
            acc = zf
            for cc in range(N_CLS):
                acc = acc + jnp.where(
                    li16 == cc, jnp.full((16,), accs[cc], jnp.float32),
                    jnp.float32(0.0))
            outrow[...] = acc
            pltpu.sync_copy(outrow, out_hbm.at[r])
            return 0

        lax.fori_loop(0, ROWS_PW, per_row, 0)

    return sck


_sc_select = _sc_make()


def kernel(x, keys, values):
    keys_pad = jnp.concatenate(
        [keys, jnp.zeros((NP - N, D), keys.dtype)], axis=0)
    s_store, mp, m2p3 = _tc_scores(x, keys_pad)
    m2p = jnp.swapaxes(m2p3, 0, 1).reshape(B, NGRP)

    m2_pad = jnp.concatenate(
        [m2p, jnp.full((B, NGP - NGRP), np.int32(-2147483648), jnp.int32)],
        axis=1)
    v_store = jnp.concatenate(
        [values, jnp.zeros((NP - N,), values.dtype)], axis=0)

    logits16 = _sc_select(
        m2_pad.reshape(B, NGP // 16, 16),
        mp.reshape(B, NGRP, 16),
        s_store.reshape(B, NWIN, 16),
        v_store.reshape(NWIN, 16),
    )
    return logits16[:, :N_CLS]
